# baseline (speedup 1.0000x reference)
import functools

import jax
import jax.numpy as jnp
from jax import lax
from jax.experimental import pallas as pl
from jax.experimental.pallas import tpu as pltpu
from jax.experimental.pallas import tpu_sc as plsc

N = 10000
E = 160000
C_IN = 256
C_HID = 256
C_OUT = 40
C2 = 128

BN = 1000
NB = N // BN
CH = C_HID // 2
NPAD = 10240

NSC = 2
NTS = 16
K = 64
EPAD = 163840
ECH = EPAD // K
EPT1 = EPAD // NTS
NC1 = EPT1 // K
EPT2 = EPAD // (NSC * NTS)
NC2 = EPT2 // K
CA = 80
CB = (ECH - CA * NTS) // NTS
CMX = max(CA, CB)
APT = NPAD // NTS
NWB = APT // K

_SC_MESH = plsc.VectorSubcoreMesh(core_axis_name="c", subcore_axis_name="s")
_SC_MESH1 = plsc.VectorSubcoreMesh(
    core_axis_name="c", subcore_axis_name="s", num_cores=1)


def _dis_from_degp(degp_blk):
    return lax.rsqrt(degp_blk[:, 0] + degp_blk[:, 1] + 1.0)


def _tc_a_body(degp, x, w1, yws):
    dis = _dis_from_degp(degp[...])
    xw = jnp.dot(x[...], w1[...], preferred_element_type=jnp.float32)
    yws[...] = (xw * dis[:, None])[None]


def _tc_a(degp, x, W1):
    return pl.pallas_call(
        _tc_a_body,
        grid=(2, NB),
        in_specs=[
            pl.BlockSpec((BN, 2), lambda c, i: (i, 0)),
            pl.BlockSpec((BN, C_IN), lambda c, i: (i, 0)),
            pl.BlockSpec((C_IN, CH), lambda c, i: (0, c)),
        ],
        out_specs=pl.BlockSpec((1, BN, CH), lambda c, i: (c, i, 0)),
        out_shape=jax.ShapeDtypeStruct((NSC, N, CH), jnp.float32),
    )(degp, x, W1)


def _tc_b_body(degp, a0, a1, y0, y1, w2, b1, yw2):
    dis = _dis_from_degp(degp[...])[:, None]
    h0 = dis * (a0[0] + y0[0])
    h1 = dis * (a1[0] + y1[0])
    h = jnp.concatenate([h0, h1], axis=1) + b1[...]
    h = jnp.maximum(h, 0.0)
    hw = jnp.dot(h, w2[...], preferred_element_type=jnp.float32)
    yw2[...] = hw * dis


def _tc_b(degp, agg1, yws, W2p, b1):
    return pl.pallas_call(
        _tc_b_body,
        grid=(NB,),
        in_specs=[
            pl.BlockSpec((BN, 2), lambda i: (i, 0)),
            pl.BlockSpec((1, BN, CH), lambda i: (0, i, 0)),
            pl.BlockSpec((1, BN, CH), lambda i: (1, i, 0)),
            pl.BlockSpec((1, BN, CH), lambda i: (0, i, 0)),
            pl.BlockSpec((1, BN, CH), lambda i: (1, i, 0)),
            pl.BlockSpec((C_HID, C2), lambda i: (0, 0)),
            pl.BlockSpec((1, C_HID), lambda i: (0, 0)),
        ],
        out_specs=pl.BlockSpec((BN, C2), lambda i: (i, 0)),
        out_shape=jax.ShapeDtypeStruct((N, C2), jnp.float32),
    )(degp, agg1, agg1, yws, yws, W2p, b1)


def _tc_c_body(degp, p0, yw2, b2, out):
    dis = _dis_from_degp(degp[...])[:, None]
    full = dis * (p0[...] + yw2[...])
    out[...] = full[:, :C_OUT] + b2[...]


def _tc_c(degp, agg2p, yw2, b2):
    return pl.pallas_call(
        _tc_c_body,
        grid=(NB,),
        in_specs=[
            pl.BlockSpec((BN, 2), lambda i: (i, 0)),
            pl.BlockSpec((BN, C2), lambda i: (i, 0)),
            pl.BlockSpec((BN, C2), lambda i: (i, 0)),
            pl.BlockSpec((1, C_OUT), lambda i: (0, 0)),
        ],
        out_specs=pl.BlockSpec((BN, C_OUT), lambda i: (i, 0)),
        out_shape=jax.ShapeDtypeStruct((N, C_OUT), jnp.float32),
    )(degp, agg2p, yw2, b2)


def _zero_vec(ref, nelem):
    def body(i, _):
        ref[pl.ds(i * 16, 16)] = jnp.zeros((16,), jnp.float32)
        return 0
    lax.fori_loop(0, nelem // 16, body, 0)


def _zero_2d(ref, rows, cols):
    def body(i, _):
        def col(k, _2):
            ref[i, pl.ds(k * 16, 16)] = jnp.zeros((16,), jnp.float32)
            return 0
        lax.fori_loop(0, cols // 16, col, 0)
        return 0
    lax.fori_loop(0, rows, body, 0)


@functools.partial(
    pl.kernel, mesh=_SC_MESH,
    out_type=jax.ShapeDtypeStruct((NSC * NPAD,), jnp.float32),
    scratch_types=[
        pltpu.VMEM((NC2, K), jnp.int32),
        pltpu.VMEM((APT,), jnp.float32),
        pltpu.VMEM((K,), jnp.float32),
        pltpu.VMEM_SHARED((NPAD,), jnp.float32),
    ])
def _sc_deg(dst2d, out, dst_v, zb, ones_v, hist):
    c = lax.axis_index("c")
    s = lax.axis_index("s")
    w = c * NTS + s
    pltpu.sync_copy(dst2d.at[pl.ds(w * NC2, NC2)], dst_v)
    _zero_vec(zb, APT)

    def fill1(i, _):
        ones_v[pl.ds(i * 16, 16)] = jnp.ones((16,), jnp.float32)
        return 0
    lax.fori_loop(0, K // 16, fill1, 0)

    pltpu.sync_copy(zb, hist.at[pl.ds(s * APT, APT)])
    plsc.subcore_barrier()

    def body(j, _):
        pltpu.sync_copy(ones_v, hist.at[dst_v.at[j]], add=True)
        return 0
    lax.fori_loop(0, NC2, body, 0)

    plsc.subcore_barrier()
    pltpu.sync_copy(hist.at[pl.ds(s * APT, APT)], zb)
    pltpu.sync_copy(zb, out.at[pl.ds(c * NPAD + s * APT, APT)])


@functools.partial(
    pl.kernel, mesh=_SC_MESH,
    out_type=jax.ShapeDtypeStruct((NSC, NPAD, CH), jnp.float32),
    scratch_types=[
        pltpu.VMEM((EPT1,), jnp.int32),
        pltpu.VMEM((NC1, K), jnp.int32),
        pltpu.VMEM((K, CH), jnp.float32),
        pltpu.VMEM((K, CH), jnp.float32),
        pltpu.VMEM_SHARED((NPAD, CH), jnp.float32),
        pltpu.SemaphoreType.DMA,
        pltpu.SemaphoreType.DMA,
    ])
def _sc_agg1(yws, srcl, dst2d, out, src_v, dst_v, ga, gb, acc, sa, sb):
    c = lax.axis_index("c")
    s = lax.axis_index("s")
    pltpu.sync_copy(srcl.at[pl.ds(s * EPT1, EPT1)], src_v)
    off = jnp.full((16,), c * N, jnp.int32)

    def addoff(i, _):
        src_v[pl.ds(i * 16, 16)] = src_v[pl.ds(i * 16, 16)] + off
        return 0
    lax.fori_loop(0, EPT1 // 16, addoff, 0)

    pltpu.sync_copy(dst2d.at[pl.ds(s * NC1, NC1)], dst_v)

    _zero_2d(ga, K, CH)

    def zacc(t, _):
        pltpu.sync_copy(ga, acc.at[pl.ds(s * APT + t * K, K)])
        return 0
    lax.fori_loop(0, NWB, zacc, 0)
    plsc.subcore_barrier()

    def gath(j, buf, sem):
        return pltpu.make_async_copy(
            yws.at[src_v.at[pl.ds(j * K, K)]], buf, sem)

    gath(0, ga, sa).start()

    def pair(p, _):
        j0 = 2 * p
        gath(j0 + 1, gb, sb).start()
        gath(j0, ga, sa).wait()
        pltpu.sync_copy(ga, acc.at[dst_v.at[j0]], add=True)

        @pl.when(j0 + 2 < NC1)
        def _():
            gath(j0 + 2, ga, sa).start()

        gath(j0 + 1, gb, sb).wait()
        pltpu.sync_copy(gb, acc.at[dst_v.at[j0 + 1]], add=True)
        return 0
    lax.fori_loop(0, NC1 // 2, pair, 0)

    plsc.subcore_barrier()

    def wb(t, _):
        pltpu.sync_copy(acc.at[pl.ds(s * APT + t * K, K)], ga)
        pltpu.sync_copy(ga, out.at[c, pl.ds(s * APT + t * K, K)])
        return 0
    lax.fori_loop(0, NWB, wb, 0)


def _agg2_half(yw2, srcl, dst2d, src_v, dst_v, ga, gb, acc, sa, sb,
               base_chunk, nchunks, table_off):
    ne = nchunks * K
    pltpu.sync_copy(srcl.at[pl.ds(base_chunk * K, ne)],
                    src_v.at[pl.ds(0, ne)])
    off = jnp.full((16,), table_off, jnp.int32)

    def addoff(i, _):
        src_v[pl.ds(i * 16, 16)] = src_v[pl.ds(i * 16, 16)] + off
        return 0
    lax.fori_loop(0, ne // 16, addoff, 0)

    pltpu.sync_copy(dst2d.at[pl.ds(base_chunk, nchunks)],
                    dst_v.at[pl.ds(0, nchunks)])

    def gath(j, buf, sem):
        return pltpu.make_async_copy(
            yw2.at[src_v.at[pl.ds(j * K, K)]], buf, sem)

    gath(0, ga, sa).start()

    def pair(p, _):
        j0 = 2 * p
        gath(j0 + 1, gb, sb).start()
        gath(j0, ga, sa).wait()
        pltpu.sync_copy(ga, acc.at[dst_v.at[j0]], add=True)

        @pl.when(j0 + 2 < nchunks)
        def _():
            gath(j0 + 2, ga, sa).start()

        gath(j0 + 1, gb, sb).wait()
        pltpu.sync_copy(gb, acc.at[dst_v.at[j0 + 1]], add=True)
        return 0
    lax.fori_loop(0, nchunks // 2, pair, 0)


@functools.partial(
    pl.kernel, mesh=_SC_MESH1,
    out_type=jax.ShapeDtypeStruct((NPAD, C2), jnp.float32),
    scratch_types=[
        pltpu.VMEM((EPT1,), jnp.int32),
        pltpu.VMEM((NC1, K), jnp.int32),
        pltpu.VMEM((K, C2), jnp.float32),
        pltpu.VMEM((K, C2), jnp.float32),
        pltpu.VMEM_SHARED((NPAD, C2), jnp.float32),
        pltpu.SemaphoreType.DMA,
        pltpu.SemaphoreType.DMA,
    ])
def _sc_agg2(yw2, srcl, dst2d, out, src_v, dst_v, ga, gb, acc, sa, sb):
    s = lax.axis_index("s")

    _zero_2d(ga, K, C2)

    def zacc(t, _):
        pltpu.sync_copy(ga, acc.at[pl.ds(s * APT + t * K, K)])
        return 0
    lax.fori_loop(0, NWB, zacc, 0)
    plsc.subcore_barrier()

    _agg2_half(yw2, srcl, dst2d, src_v, dst_v, ga, gb, acc, sa, sb,
               s * NC1, NC1, 0)

    plsc.subcore_barrier()

    def wb(t, _):
        pltpu.sync_copy(acc.at[pl.ds(s * APT + t * K, K)], ga)
        pltpu.sync_copy(ga, out.at[pl.ds(s * APT + t * K, K)])
        return 0
    lax.fori_loop(0, NWB, wb, 0)


@jax.jit
def kernel(x, edge_index, W1, b1, W2, b2):
    srcp = jnp.concatenate(
        [edge_index[0], jnp.zeros((EPAD - E,), jnp.int32)])
    pad_dst = N + jnp.arange(EPAD - E, dtype=jnp.int32) % (NPAD - N)
    dst2d = jnp.concatenate([edge_index[1], pad_dst]).reshape(ECH, K)
    b1r = b1.reshape(1, C_HID)
    b2r = b2.reshape(1, C_OUT)
    W2p = jnp.zeros((C_HID, C2), jnp.float32).at[:, :C_OUT].set(W2)

    degf = _sc_deg(dst2d)
    degp = degf.reshape(NSC, NPAD)[:, :N].T
    yws = _tc_a(degp, x, W1)
    yws_flat = yws.reshape(NSC * N, CH)
    agg1 = _sc_agg1(yws_flat, srcp, dst2d)
    yw2 = _tc_b(degp, agg1, yws, W2p, b1r)
    agg2p = _sc_agg2(yw2, srcp, dst2d)
    return _tc_c(degp, agg2p[:N], yw2, b2r)


# speedup vs baseline: 1.0908x; 1.0908x over previous
import functools

import jax
import jax.numpy as jnp
from jax import lax
from jax.experimental import pallas as pl
from jax.experimental.pallas import tpu as pltpu
from jax.experimental.pallas import tpu_sc as plsc

N = 10000
E = 160000
C_IN = 256
C_HID = 256
C_OUT = 40
C2 = 128

BN = 1000
NB = N // BN
CH = C_HID // 2
NPAD = 10240

NSC = 2
NTS = 16
K = 64
EPAD = 163840
ECH = EPAD // K
EPT1 = EPAD // NTS
NC1 = EPT1 // K
EPT2 = EPAD // (NSC * NTS)
NC2 = EPT2 // K
CA = 80
CB = (ECH - CA * NTS) // NTS
CMX = max(CA, CB)
APT = NPAD // NTS
NWB = APT // K

_SC_MESH = plsc.VectorSubcoreMesh(core_axis_name="c", subcore_axis_name="s")
_SC_MESH1 = plsc.VectorSubcoreMesh(
    core_axis_name="c", subcore_axis_name="s", num_cores=1)


def _dis_from_degp(degp_blk):
    return lax.rsqrt(degp_blk[:, 0] + degp_blk[:, 1] + 1.0)


def _tc_a_body(degp, x, w1, yws):
    dis = _dis_from_degp(degp[...])
    xw = jnp.dot(x[...], w1[...], preferred_element_type=jnp.float32)
    yws[...] = (xw * dis[:, None])[None]


def _tc_a(degp, x, W1):
    return pl.pallas_call(
        _tc_a_body,
        grid=(2, NB),
        in_specs=[
            pl.BlockSpec((BN, 2), lambda c, i: (i, 0)),
            pl.BlockSpec((BN, C_IN), lambda c, i: (i, 0)),
            pl.BlockSpec((C_IN, CH), lambda c, i: (0, c)),
        ],
        out_specs=pl.BlockSpec((1, BN, CH), lambda c, i: (c, i, 0)),
        out_shape=jax.ShapeDtypeStruct((NSC, N, CH), jnp.float32),
    )(degp, x, W1)


def _tc_b_body(degp, a0, a1, y0, y1, w2, b1, yw2):
    dis = _dis_from_degp(degp[...])[:, None]
    h0 = dis * (a0[0] + y0[0])
    h1 = dis * (a1[0] + y1[0])
    h = jnp.concatenate([h0, h1], axis=1) + b1[...]
    h = jnp.maximum(h, 0.0)
    hw = jnp.dot(h, w2[...], preferred_element_type=jnp.float32)
    yw2[...] = hw * dis


def _tc_b(degp, agg1, yws, W2p, b1):
    return pl.pallas_call(
        _tc_b_body,
        grid=(NB,),
        in_specs=[
            pl.BlockSpec((BN, 2), lambda i: (i, 0)),
            pl.BlockSpec((1, BN, CH), lambda i: (0, i, 0)),
            pl.BlockSpec((1, BN, CH), lambda i: (1, i, 0)),
            pl.BlockSpec((1, BN, CH), lambda i: (0, i, 0)),
            pl.BlockSpec((1, BN, CH), lambda i: (1, i, 0)),
            pl.BlockSpec((C_HID, C2), lambda i: (0, 0)),
            pl.BlockSpec((1, C_HID), lambda i: (0, 0)),
        ],
        out_specs=pl.BlockSpec((BN, C2), lambda i: (i, 0)),
        out_shape=jax.ShapeDtypeStruct((N, C2), jnp.float32),
    )(degp, agg1, agg1, yws, yws, W2p, b1)


def _tc_c_body(degp, p0, p1, yw2, b2, out):
    dis = _dis_from_degp(degp[...])[:, None]
    full = dis * (p0[0] + p1[0] + yw2[...])
    out[...] = full[:, :C_OUT] + b2[...]


def _tc_c(degp, agg2p, yw2, b2):
    return pl.pallas_call(
        _tc_c_body,
        grid=(NB,),
        in_specs=[
            pl.BlockSpec((BN, 2), lambda i: (i, 0)),
            pl.BlockSpec((1, BN, C2), lambda i: (0, i, 0)),
            pl.BlockSpec((1, BN, C2), lambda i: (1, i, 0)),
            pl.BlockSpec((BN, C2), lambda i: (i, 0)),
            pl.BlockSpec((1, C_OUT), lambda i: (0, 0)),
        ],
        out_specs=pl.BlockSpec((BN, C_OUT), lambda i: (i, 0)),
        out_shape=jax.ShapeDtypeStruct((N, C_OUT), jnp.float32),
    )(degp, agg2p, agg2p, yw2, b2)


def _zero_vec(ref, nelem):
    def body(i, _):
        ref[pl.ds(i * 16, 16)] = jnp.zeros((16,), jnp.float32)
        return 0
    lax.fori_loop(0, nelem // 16, body, 0)


def _zero_2d(ref, rows, cols):
    def body(i, _):
        def col(k, _2):
            ref[i, pl.ds(k * 16, 16)] = jnp.zeros((16,), jnp.float32)
            return 0
        lax.fori_loop(0, cols // 16, col, 0)
        return 0
    lax.fori_loop(0, rows, body, 0)


@functools.partial(
    pl.kernel, mesh=_SC_MESH,
    out_type=jax.ShapeDtypeStruct((NSC * NPAD,), jnp.float32),
    scratch_types=[
        pltpu.VMEM((NC2, K), jnp.int32),
        pltpu.VMEM((APT,), jnp.float32),
        pltpu.VMEM((K,), jnp.float32),
        pltpu.VMEM_SHARED((NPAD,), jnp.float32),
    ])
def _sc_deg(dst2d, out, dst_v, zb, ones_v, hist):
    c = lax.axis_index("c")
    s = lax.axis_index("s")
    w = c * NTS + s
    pltpu.sync_copy(dst2d.at[pl.ds(w * NC2, NC2)], dst_v)
    _zero_vec(zb, APT)

    def fill1(i, _):
        ones_v[pl.ds(i * 16, 16)] = jnp.ones((16,), jnp.float32)
        return 0
    lax.fori_loop(0, K // 16, fill1, 0)

    pltpu.sync_copy(zb, hist.at[pl.ds(s * APT, APT)])
    plsc.subcore_barrier()

    def body(j, _):
        pltpu.sync_copy(ones_v, hist.at[dst_v.at[j]], add=True)
        return 0
    lax.fori_loop(0, NC2, body, 0)

    plsc.subcore_barrier()
    pltpu.sync_copy(hist.at[pl.ds(s * APT, APT)], zb)
    pltpu.sync_copy(zb, out.at[pl.ds(c * NPAD + s * APT, APT)])


@functools.partial(
    pl.kernel, mesh=_SC_MESH,
    out_type=jax.ShapeDtypeStruct((NSC, NPAD, CH), jnp.float32),
    scratch_types=[
        pltpu.VMEM((EPT1,), jnp.int32),
        pltpu.VMEM((NC1, K), jnp.int32),
        pltpu.VMEM((K, CH), jnp.float32),
        pltpu.VMEM((K, CH), jnp.float32),
        pltpu.VMEM_SHARED((NPAD, CH), jnp.float32),
        pltpu.SemaphoreType.DMA,
        pltpu.SemaphoreType.DMA,
    ])
def _sc_agg1(yws, srcl, dst2d, out, src_v, dst_v, ga, gb, acc, sa, sb):
    c = lax.axis_index("c")
    s = lax.axis_index("s")
    pltpu.sync_copy(srcl.at[pl.ds(s * EPT1, EPT1)], src_v)
    off = jnp.full((16,), c * N, jnp.int32)

    def addoff(i, _):
        src_v[pl.ds(i * 16, 16)] = src_v[pl.ds(i * 16, 16)] + off
        return 0
    lax.fori_loop(0, EPT1 // 16, addoff, 0)

    pltpu.sync_copy(dst2d.at[pl.ds(s * NC1, NC1)], dst_v)

    _zero_2d(ga, K, CH)

    def zacc(t, _):
        pltpu.sync_copy(ga, acc.at[pl.ds(s * APT + t * K, K)])
        return 0
    lax.fori_loop(0, NWB, zacc, 0)
    plsc.subcore_barrier()

    def gath(j, buf, sem):
        return pltpu.make_async_copy(
            yws.at[src_v.at[pl.ds(j * K, K)]], buf, sem)

    gath(0, ga, sa).start()

    def pair(p, _):
        j0 = 2 * p
        gath(j0 + 1, gb, sb).start()
        gath(j0, ga, sa).wait()
        pltpu.sync_copy(ga, acc.at[dst_v.at[j0]], add=True)

        @pl.when(j0 + 2 < NC1)
        def _():
            gath(j0 + 2, ga, sa).start()

        gath(j0 + 1, gb, sb).wait()
        pltpu.sync_copy(gb, acc.at[dst_v.at[j0 + 1]], add=True)
        return 0
    lax.fori_loop(0, NC1 // 2, pair, 0)

    plsc.subcore_barrier()

    def wb(t, _):
        pltpu.sync_copy(acc.at[pl.ds(s * APT + t * K, K)], ga)
        pltpu.sync_copy(ga, out.at[c, pl.ds(s * APT + t * K, K)])
        return 0
    lax.fori_loop(0, NWB, wb, 0)


def _agg2_half(yw2, srcl, dst2d, src_v, dst_v, ga, gb, acc, sa, sb,
               base_chunk, nchunks, table_off):
    ne = nchunks * K
    pltpu.sync_copy(srcl.at[pl.ds(base_chunk * K, ne)],
                    src_v.at[pl.ds(0, ne)])
    off = jnp.full((16,), table_off, jnp.int32)

    def addoff(i, _):
        src_v[pl.ds(i * 16, 16)] = src_v[pl.ds(i * 16, 16)] + off
        return 0
    lax.fori_loop(0, ne // 16, addoff, 0)

    pltpu.sync_copy(dst2d.at[pl.ds(base_chunk, nchunks)],
                    dst_v.at[pl.ds(0, nchunks)])

    def gath(j, buf, sem):
        return pltpu.make_async_copy(
            yw2.at[src_v.at[pl.ds(j * K, K)]], buf, sem)

    gath(0, ga, sa).start()

    def pair(p, _):
        j0 = 2 * p
        gath(j0 + 1, gb, sb).start()
        gath(j0, ga, sa).wait()
        pltpu.sync_copy(ga, acc.at[dst_v.at[j0]], add=True)

        @pl.when(j0 + 2 < nchunks)
        def _():
            gath(j0 + 2, ga, sa).start()

        gath(j0 + 1, gb, sb).wait()
        pltpu.sync_copy(gb, acc.at[dst_v.at[j0 + 1]], add=True)
        return 0
    lax.fori_loop(0, nchunks // 2, pair, 0)


@functools.partial(
    pl.kernel, mesh=_SC_MESH,
    out_type=jax.ShapeDtypeStruct((NSC, NPAD, C2), jnp.float32),
    scratch_types=[
        pltpu.VMEM((EPT2,), jnp.int32),
        pltpu.VMEM((NC2, K), jnp.int32),
        pltpu.VMEM((K, C2), jnp.float32),
        pltpu.VMEM((K, C2), jnp.float32),
        pltpu.VMEM_SHARED((NPAD, C2), jnp.float32),
        pltpu.SemaphoreType.DMA,
        pltpu.SemaphoreType.DMA,
    ])
def _sc_agg2(yw2, srcl, dst2d, out, src_v, dst_v, ga, gb, acc, sa, sb):
    c = lax.axis_index("c")
    s = lax.axis_index("s")
    w = c * NTS + s

    _zero_2d(ga, K, C2)

    def zacc(t, _):
        pltpu.sync_copy(ga, acc.at[pl.ds(s * APT + t * K, K)])
        return 0
    lax.fori_loop(0, NWB, zacc, 0)
    plsc.subcore_barrier()

    _agg2_half(yw2, srcl, dst2d, src_v, dst_v, ga, gb, acc, sa, sb,
               w * NC2, NC2, 0)

    plsc.subcore_barrier()

    def wb(t, _):
        pltpu.sync_copy(acc.at[pl.ds(s * APT + t * K, K)], ga)
        pltpu.sync_copy(ga, out.at[c, pl.ds(s * APT + t * K, K)])
        return 0
    lax.fori_loop(0, NWB, wb, 0)


@jax.jit
def kernel(x, edge_index, W1, b1, W2, b2):
    srcp = jnp.concatenate(
        [edge_index[0], jnp.zeros((EPAD - E,), jnp.int32)])
    pad_dst = N + jnp.arange(EPAD - E, dtype=jnp.int32) % (NPAD - N)
    dst2d = jnp.concatenate([edge_index[1], pad_dst]).reshape(ECH, K)
    b1r = b1.reshape(1, C_HID)
    b2r = b2.reshape(1, C_OUT)
    W2p = jnp.zeros((C_HID, C2), jnp.float32).at[:, :C_OUT].set(W2)

    degf = _sc_deg(dst2d)
    degp = degf.reshape(NSC, NPAD)[:, :N].T
    yws = _tc_a(degp, x, W1)
    yws_flat = yws.reshape(NSC * N, CH)
    agg1 = _sc_agg1(yws_flat, srcp, dst2d)
    yw2 = _tc_b(degp, agg1, yws, W2p, b1r)
    agg2p = _sc_agg2(yw2, srcp, dst2d)
    return _tc_c(degp, agg2p, yw2, b2r)
